# baseline (speedup 1.0000x reference)
import functools

import jax
import jax.numpy as jnp
from jax import lax
from jax.experimental import pallas as pl
from jax.experimental.pallas import tpu as pltpu

_NEG = -jnp.inf
_K = 5
_CW = 128


def _body(y_ref, x_ref, o_ref, r_scr, sy_ref, *, n, bb, cb, nc, b_total):
    i = pl.program_id(0)
    j = pl.program_id(1)
    cpb = cb // _CW

    @pl.when(j == 0)
    def _init():
        r_scr[...] = jnp.full_like(r_scr, _NEG)
        sy_ref[...] = jnp.zeros_like(sy_ref)

    @pl.when((j == 0) & (i == 0))
    def _init_out():
        o_ref[...] = jnp.zeros((1, 1), jnp.float32)

    xb = x_ref[...]
    col = j * cb + lax.broadcasted_iota(jnp.int32, (bb, cb), 1)
    yb = y_ref[0]
    eq_lab = col == yb
    sy_ref[...] += jnp.sum(jnp.where(eq_lab, xb, 0.0), axis=1, keepdims=True)
    xm = jnp.where(eq_lab | (col >= n), _NEG, xb)

    rs = [r_scr[:, s * _CW:(s + 1) * _CW] for s in range(_K)]
    for k in range(cpb):
        m = xm[:, k * _CW:(k + 1) * _CW]
        for s in range(_K):
            hi = jnp.maximum(rs[s], m)
            m = jnp.minimum(rs[s], m)
            rs[s] = hi
    for s in range(_K):
        r_scr[:, s * _CW:(s + 1) * _CW] = rs[s]

    @pl.when(j == nc - 1)
    def _final():
        cand = jnp.concatenate(rs, axis=1)
        cur = cand
        zero = jnp.zeros((bb, 1), jnp.float32)
        cum = zero
        sum5 = zero
        t5 = zero
        for t in range(_K):
            mt = jnp.max(cur, axis=1, keepdims=True)
            eq = cur == mt
            ct = jnp.sum(eq.astype(jnp.float32), axis=1, keepdims=True)
            if t < _K - 1:
                cur = jnp.where(eq, _NEG, cur)
            take = jnp.clip(5.0 - cum, 0.0, ct)
            sum5 += mt * take
            crossed = (cum < 5.0) & (cum + ct >= 5.0)
            t5 += jnp.where(crossed, mt, 0.0)
            cum += ct
        sy = sy_ref[:, 0:1]
        sum5p = sum5 + 5.0
        min6 = jnp.minimum(t5 + 1.0, sy)
        tot = sum5p + sy - min6
        loss = jnp.maximum(tot / 5.0 - sy, 0.0)
        o_ref[...] += jnp.sum(loss).reshape(1, 1) / b_total


def kernel(x, y):
    b, n = x.shape
    bb = 1024 if b % 1024 == 0 else b
    nr = b // bb
    cb = 2048
    nc = (n + cb - 1) // cb
    y3 = y.reshape(nr, bb, 1)

    out = pl.pallas_call(
        functools.partial(_body, n=n, bb=bb, cb=cb, nc=nc, b_total=b),
        grid=(nr, nc),
        in_specs=[
            pl.BlockSpec((1, bb, 1), lambda i, j: (i, 0, 0)),
            pl.BlockSpec((bb, cb), lambda i, j: (i, j)),
        ],
        out_specs=pl.BlockSpec((1, 1), lambda i, j: (0, 0)),
        out_shape=jax.ShapeDtypeStruct((1, 1), jnp.float32),
        scratch_shapes=[
            pltpu.VMEM((bb, _K * _CW), jnp.float32),
            pltpu.VMEM((bb, 8), jnp.float32),
        ],
    )(y3, x)
    return out[0, 0]


# speedup vs baseline: 1.4736x; 1.4736x over previous
import functools

import jax
import jax.numpy as jnp
from jax import lax
from jax.experimental import pallas as pl
from jax.experimental.pallas import tpu as pltpu

_NEG = -jnp.inf
_K = 5
_CW = 128


def _body(y_ref, x_ref, o_ref, r_scr, sy_ref, *, n, bb, cb, nc, b_total):
    i = pl.program_id(0)
    j = pl.program_id(1)
    cpb = cb // _CW

    @pl.when(j == 0)
    def _init():
        r_scr[...] = jnp.full_like(r_scr, _NEG)
        sy_ref[...] = jnp.zeros_like(sy_ref)

    @pl.when((j == 0) & (i == 0))
    def _init_out():
        o_ref[...] = jnp.zeros((1, 1), jnp.float32)

    xb = x_ref[...]
    col = j * cb + lax.broadcasted_iota(jnp.int32, (bb, cb), 1)
    yb = y_ref[0]
    eq_lab = col == yb
    sy_ref[...] += jnp.sum(jnp.where(eq_lab, xb, 0.0), axis=1, keepdims=True)
    xm = jnp.where(eq_lab | (col >= n), _NEG, xb)

    rs = [r_scr[:, s * _CW:(s + 1) * _CW] for s in range(_K)]
    for k4 in range(cpb // 4):
        a = xm[:, (4 * k4 + 0) * _CW:(4 * k4 + 1) * _CW]
        b2 = xm[:, (4 * k4 + 1) * _CW:(4 * k4 + 2) * _CW]
        c = xm[:, (4 * k4 + 2) * _CW:(4 * k4 + 3) * _CW]
        d = xm[:, (4 * k4 + 3) * _CW:(4 * k4 + 4) * _CW]
        h1 = jnp.maximum(a, b2)
        l1 = jnp.minimum(a, b2)
        h2 = jnp.maximum(c, d)
        l2 = jnp.minimum(c, d)
        s0 = jnp.maximum(h1, h2)
        t = jnp.minimum(h1, h2)
        s3 = jnp.minimum(l1, l2)
        u = jnp.maximum(l1, l2)
        s1 = jnp.maximum(t, u)
        s2 = jnp.minimum(t, u)
        c0 = rs[0]
        c1 = jnp.maximum(rs[1], s3)
        c2 = jnp.maximum(rs[2], s2)
        c3 = jnp.maximum(rs[3], s1)
        c4 = jnp.maximum(rs[4], s0)
        a0 = jnp.maximum(c0, c4)
        b0 = jnp.minimum(c0, c4)
        d1 = jnp.maximum(c1, c3)
        d3 = jnp.minimum(c1, c3)
        m12 = jnp.maximum(c2, d3)
        n12 = jnp.minimum(c2, d3)
        e1 = jnp.maximum(d1, b0)
        r = jnp.minimum(d1, b0)
        e2 = jnp.maximum(m12, r)
        r2 = jnp.minimum(m12, r)
        e3 = jnp.maximum(n12, r2)
        e4 = jnp.minimum(n12, r2)
        rs = [a0, e1, e2, e3, e4]
    for s in range(_K):
        r_scr[:, s * _CW:(s + 1) * _CW] = rs[s]

    @pl.when(j == nc - 1)
    def _final():
        cand = jnp.concatenate(rs, axis=1)
        cur = cand
        zero = jnp.zeros((bb, 1), jnp.float32)
        cum = zero
        sum5 = zero
        t5 = zero
        for t in range(_K):
            mt = jnp.max(cur, axis=1, keepdims=True)
            eq = cur == mt
            ct = jnp.sum(eq.astype(jnp.float32), axis=1, keepdims=True)
            if t < _K - 1:
                cur = jnp.where(eq, _NEG, cur)
            take = jnp.clip(5.0 - cum, 0.0, ct)
            sum5 += mt * take
            crossed = (cum < 5.0) & (cum + ct >= 5.0)
            t5 += jnp.where(crossed, mt, 0.0)
            cum += ct
        sy = sy_ref[:, 0:1]
        sum5p = sum5 + 5.0
        min6 = jnp.minimum(t5 + 1.0, sy)
        tot = sum5p + sy - min6
        loss = jnp.maximum(tot / 5.0 - sy, 0.0)
        o_ref[...] += jnp.sum(loss).reshape(1, 1) / b_total


def kernel(x, y):
    b, n = x.shape
    bb = 1024 if b % 1024 == 0 else b
    nr = b // bb
    cb = 2048
    nc = (n + cb - 1) // cb
    y3 = y.reshape(nr, bb, 1)

    out = pl.pallas_call(
        functools.partial(_body, n=n, bb=bb, cb=cb, nc=nc, b_total=b),
        grid=(nr, nc),
        in_specs=[
            pl.BlockSpec((1, bb, 1), lambda i, j: (i, 0, 0)),
            pl.BlockSpec((bb, cb), lambda i, j: (i, j)),
        ],
        out_specs=pl.BlockSpec((1, 1), lambda i, j: (0, 0)),
        out_shape=jax.ShapeDtypeStruct((1, 1), jnp.float32),
        scratch_shapes=[
            pltpu.VMEM((bb, _K * _CW), jnp.float32),
            pltpu.VMEM((bb, 8), jnp.float32),
        ],
    )(y3, x)
    return out[0, 0]
